# Initial kernel scaffold; baseline (speedup 1.0000x reference)
import jax
import jax.numpy as jnp
from jax.experimental import pallas as pl


def kernel(x, support, W0, b0, W1, b1, Wp, bp):
    raise NotImplementedError("write your pallas kernel here")


# speedup vs baseline: 1.0051x; 1.0051x over previous
import jax
import jax.numpy as jnp
from jax.experimental import pallas as pl
from jax.experimental.pallas import tpu as pltpu

_BM = 400


def _proj_kernel(x_ref, w_ref, o_ref):
    o_ref[...] = jnp.dot(x_ref[...], w_ref[...],
                         preferred_element_type=jnp.float32)


def _layer1_kernel(s_ref, g_ref, b_ref, w1_ref, wpt_ref, bp_ref,
                   g1_ref, p_ref):
    h = jnp.dot(s_ref[...], g_ref[...], preferred_element_type=jnp.float32)
    h = jnp.maximum(h + b_ref[...], 0.0)
    g1_ref[...] = jnp.dot(h, w1_ref[...], preferred_element_type=jnp.float32)
    p_ref[...] = jnp.dot(h, wpt_ref[...],
                         preferred_element_type=jnp.float32) + bp_ref[...]


def _layer2_kernel(s_ref, g_ref, b_ref, wpb_ref, p_ref, o_ref):
    h = jnp.dot(s_ref[...], g_ref[...], preferred_element_type=jnp.float32)
    h = jnp.maximum(h + b_ref[...], 0.0)
    o_ref[...] = p_ref[...] + jnp.dot(h, wpb_ref[...],
                                      preferred_element_type=jnp.float32)


def kernel(x, support, W0, b0, W1, b1, Wp, bp):
    n, d_in = x.shape
    d_h = W0.shape[1]
    d_out = Wp.shape[1]
    bm = _BM
    grid = (n // bm,)

    b0r = b0.reshape(1, -1)
    b1r = b1.reshape(1, -1)
    bpr = bp.reshape(1, -1)
    wp_top = Wp[:d_h]
    wp_bot = Wp[d_h:]

    g0 = pl.pallas_call(
        _proj_kernel,
        out_shape=jax.ShapeDtypeStruct((n, d_h), jnp.float32),
    )(x, W0)

    full = lambda r, c: pl.BlockSpec((r, c), lambda i: (0, 0))
    rowblk = lambda c: pl.BlockSpec((bm, c), lambda i: (i, 0))

    g1, p = pl.pallas_call(
        _layer1_kernel,
        grid=grid,
        in_specs=[
            rowblk(n),
            full(n, d_h),
            full(1, d_h),
            full(d_h, d_h),
            full(d_h, d_out),
            full(1, d_out),
        ],
        out_specs=[rowblk(d_h), rowblk(d_out)],
        out_shape=[
            jax.ShapeDtypeStruct((n, d_h), jnp.float32),
            jax.ShapeDtypeStruct((n, d_out), jnp.float32),
        ],
        compiler_params=pltpu.CompilerParams(
            dimension_semantics=("parallel",)),
    )(support, g0, b0r, W1, wp_top, bpr)

    out = pl.pallas_call(
        _layer2_kernel,
        grid=grid,
        in_specs=[
            rowblk(n),
            full(n, d_h),
            full(1, d_h),
            full(d_h, d_out),
            rowblk(d_out),
        ],
        out_specs=rowblk(d_out),
        out_shape=jax.ShapeDtypeStruct((n, d_out), jnp.float32),
        compiler_params=pltpu.CompilerParams(
            dimension_semantics=("parallel",)),
    )(support, g1, b1r, wp_bot, p)

    return out


# speedup vs baseline: 1.0178x; 1.0126x over previous
import jax
import jax.numpy as jnp
from jax.experimental import pallas as pl
from jax.experimental.pallas import tpu as pltpu

_BM = 400


def _proj_kernel(x_ref, w_ref, o_ref):
    o_ref[...] = jnp.dot(x_ref[...], w_ref[...],
                         preferred_element_type=jnp.float32
                         ).astype(jnp.bfloat16)


def _layer1_kernel(s_ref, g_ref, b_ref, w1_ref, wpt_ref, bp_ref,
                   g1_ref, p_ref):
    s = s_ref[...].astype(jnp.bfloat16)
    h = jnp.dot(s, g_ref[...], preferred_element_type=jnp.float32)
    h = jnp.maximum(h + b_ref[...], 0.0)
    g1_ref[...] = jnp.dot(h, w1_ref[...],
                          preferred_element_type=jnp.float32
                          ).astype(jnp.bfloat16)
    p_ref[...] = jnp.dot(h, wpt_ref[...],
                         preferred_element_type=jnp.float32) + bp_ref[...]


def _layer2_kernel(s_ref, g_ref, b_ref, wpb_ref, p_ref, o_ref):
    s = s_ref[...].astype(jnp.bfloat16)
    h = jnp.dot(s, g_ref[...], preferred_element_type=jnp.float32)
    h = jnp.maximum(h + b_ref[...], 0.0)
    o_ref[...] = p_ref[...] + jnp.dot(h, wpb_ref[...],
                                      preferred_element_type=jnp.float32)


def kernel(x, support, W0, b0, W1, b1, Wp, bp):
    n, d_in = x.shape
    d_h = W0.shape[1]
    d_out = Wp.shape[1]
    bm = _BM
    grid = (n // bm,)

    b0r = b0.reshape(1, -1)
    b1r = b1.reshape(1, -1)
    bpr = bp.reshape(1, -1)
    wp_top = Wp[:d_h]
    wp_bot = Wp[d_h:]

    g0 = pl.pallas_call(
        _proj_kernel,
        out_shape=jax.ShapeDtypeStruct((n, d_h), jnp.bfloat16),
    )(x, W0)

    full = lambda r, c: pl.BlockSpec((r, c), lambda i: (0, 0))
    rowblk = lambda c: pl.BlockSpec((bm, c), lambda i: (i, 0))

    g1, p = pl.pallas_call(
        _layer1_kernel,
        grid=grid,
        in_specs=[
            rowblk(n),
            full(n, d_h),
            full(1, d_h),
            full(d_h, d_h),
            full(d_h, d_out),
            full(1, d_out),
        ],
        out_specs=[rowblk(d_h), rowblk(d_out)],
        out_shape=[
            jax.ShapeDtypeStruct((n, d_h), jnp.bfloat16),
            jax.ShapeDtypeStruct((n, d_out), jnp.float32),
        ],
        compiler_params=pltpu.CompilerParams(
            dimension_semantics=("parallel",)),
    )(support, g0, b0r, W1, wp_top, bpr)

    out = pl.pallas_call(
        _layer2_kernel,
        grid=grid,
        in_specs=[
            rowblk(n),
            full(n, d_h),
            full(1, d_h),
            full(d_h, d_out),
            rowblk(d_out),
        ],
        out_specs=rowblk(d_out),
        out_shape=jax.ShapeDtypeStruct((n, d_out), jnp.float32),
        compiler_params=pltpu.CompilerParams(
            dimension_semantics=("parallel",)),
    )(support, g1, b1r, wp_bot, p)

    return out


# speedup vs baseline: 1.0261x; 1.0081x over previous
import jax
import jax.numpy as jnp
from jax.experimental import pallas as pl
from jax.experimental.pallas import tpu as pltpu

_BM = 400
_NSPLIT = 2


def _proj_kernel(x_ref, w_ref, o_ref):
    o_ref[...] = jnp.dot(x_ref[...], w_ref[...],
                         preferred_element_type=jnp.float32
                         ).astype(jnp.bfloat16)


def _layer1_kernel(*refs):
    ns = _NSPLIT
    s_refs = refs[:ns]
    g_ref, b_ref, w1_ref, wpt_ref, bp_ref, g1_ref, p_ref = refs[ns:]
    sub = s_refs[0].shape[0]
    g = g_ref[...]
    for j in range(ns):
        h = jnp.dot(s_refs[j][...].astype(jnp.bfloat16), g,
                    preferred_element_type=jnp.float32)
        h = jnp.maximum(h + b_ref[...], 0.0)
        rows = pl.ds(j * sub, sub)
        g1_ref[rows, :] = jnp.dot(h, w1_ref[...],
                                  preferred_element_type=jnp.float32
                                  ).astype(jnp.bfloat16)
        p_ref[rows, :] = jnp.dot(h, wpt_ref[...],
                                 preferred_element_type=jnp.float32
                                 ) + bp_ref[...]


def _layer2_kernel(*refs):
    ns = _NSPLIT
    s_refs = refs[:ns]
    g_ref, b_ref, wpb_ref, p_ref, o_ref = refs[ns:]
    sub = s_refs[0].shape[0]
    g = g_ref[...]
    for j in range(ns):
        h = jnp.dot(s_refs[j][...].astype(jnp.bfloat16), g,
                    preferred_element_type=jnp.float32)
        h = jnp.maximum(h + b_ref[...], 0.0)
        rows = pl.ds(j * sub, sub)
        o_ref[rows, :] = p_ref[rows, :] + jnp.dot(
            h, wpb_ref[...], preferred_element_type=jnp.float32)


def kernel(x, support, W0, b0, W1, b1, Wp, bp):
    n, d_in = x.shape
    d_h = W0.shape[1]
    d_out = Wp.shape[1]
    bm = _BM
    ns = _NSPLIT
    sub = bm // ns
    grid = (n // bm,)

    b0r = b0.reshape(1, -1)
    b1r = b1.reshape(1, -1)
    bpr = bp.reshape(1, -1)
    wp_top = Wp[:d_h]
    wp_bot = Wp[d_h:]

    g0 = pl.pallas_call(
        _proj_kernel,
        out_shape=jax.ShapeDtypeStruct((n, d_h), jnp.bfloat16),
    )(x, W0)

    full = lambda r, c: pl.BlockSpec((r, c), lambda i: (0, 0))
    rowblk = lambda c: pl.BlockSpec((bm, c), lambda i: (i, 0))
    s_specs = [pl.BlockSpec((sub, n), lambda i, j=j: (i * ns + j, 0))
               for j in range(ns)]

    g1, p = pl.pallas_call(
        _layer1_kernel,
        grid=grid,
        in_specs=s_specs + [
            full(n, d_h),
            full(1, d_h),
            full(d_h, d_h),
            full(d_h, d_out),
            full(1, d_out),
        ],
        out_specs=[rowblk(d_h), rowblk(d_out)],
        out_shape=[
            jax.ShapeDtypeStruct((n, d_h), jnp.bfloat16),
            jax.ShapeDtypeStruct((n, d_out), jnp.float32),
        ],
        compiler_params=pltpu.CompilerParams(
            dimension_semantics=("parallel",)),
    )(*([support] * ns), g0, b0r, W1, wp_top, bpr)

    out = pl.pallas_call(
        _layer2_kernel,
        grid=grid,
        in_specs=s_specs + [
            full(n, d_h),
            full(1, d_h),
            full(d_h, d_out),
            rowblk(d_out),
        ],
        out_specs=rowblk(d_out),
        out_shape=jax.ShapeDtypeStruct((n, d_out), jnp.float32),
        compiler_params=pltpu.CompilerParams(
            dimension_semantics=("parallel",)),
    )(*([support] * ns), g1, b1r, wp_bot, p)

    return out
